# baseline (speedup 1.0000x reference)
import jax
import jax.numpy as jnp
from jax.experimental import pallas as pl
from jax.experimental.pallas import tpu as pltpu

N_BINS = 15
ROWS_PER_BLOCK = 200
NBUF = 16


def _ece_kernel(lo_ref, hi_ref, probs_ref, labels_ref, out_ref,
                pbuf, lbuf, psem, lsem):
    n, c = probs_ref.shape
    r = ROWS_PER_BLOCK
    nblk = n // r

    def start_copy(block, slot):
        pltpu.make_async_copy(
            probs_ref.at[pl.ds(block * r, r), :], pbuf.at[slot], psem.at[slot]
        ).start()
        pltpu.make_async_copy(
            labels_ref.at[pl.ds(block * r, r), :], lbuf.at[slot], lsem.at[slot]
        ).start()

    for b in range(NBUF):
        start_copy(b, b)

    lo = lo_ref[...]
    hi = hi_ref[...]

    def body(i, carry):
        num_p, acc_p, conf_p = carry
        slot = jax.lax.rem(i, NBUF)
        pltpu.make_async_copy(
            probs_ref.at[pl.ds(i * r, r), :], pbuf.at[slot], psem.at[slot]
        ).wait()
        pltpu.make_async_copy(
            labels_ref.at[pl.ds(i * r, r), :], lbuf.at[slot], lsem.at[slot]
        ).wait()

        x = pbuf[slot]
        lab = lbuf[slot]
        conf = jnp.max(x, axis=1, keepdims=True)
        col = jax.lax.broadcasted_iota(jnp.int32, x.shape, 1)
        pred = jnp.min(jnp.where(x == conf, col, c), axis=1, keepdims=True)
        acc = (pred == lab).astype(jnp.float32)
        onehot = ((conf > lo) & (conf <= hi)).astype(jnp.float32)

        @pl.when(i + NBUF < nblk)
        def _next():
            start_copy(i + NBUF, slot)

        return (num_p + jnp.sum(onehot, axis=0, keepdims=True),
                acc_p + jnp.sum(onehot * acc, axis=0, keepdims=True),
                conf_p + jnp.sum(onehot * conf, axis=0, keepdims=True))

    zeros = jnp.zeros((1, 128), jnp.float32)
    num, sacc, sconf = jax.lax.fori_loop(0, nblk, body, (zeros, zeros, zeros))

    safe_n = jnp.maximum(num, 1.0)
    acc_bin = sacc / safe_n
    conf_bin = sconf / safe_n
    has = num > 0.0
    ece = jnp.sum(jnp.where(has, jnp.abs(conf_bin - acc_bin) * num, 0.0))
    out_ref[0:1, :] = jnp.full_like(num, ece)
    out_ref[1:2, :] = jnp.where(has, acc_bin * num, 0.0)
    out_ref[2:3, :] = jnp.where(has, num, 0.0)


def kernel(probs, labels, mode):
    n, c = probs.shape
    r = ROWS_PER_BLOCK

    bb = jnp.linspace(0.0, 1.0, N_BINS + 1)
    lo = jnp.full((1, 128), 2.0, dtype=jnp.float32).at[0, :N_BINS].set(bb[:-1])
    hi = jnp.full((1, 128), -1.0, dtype=jnp.float32).at[0, :N_BINS].set(bb[1:])
    labels2 = labels.reshape(n, 1)

    out = pl.pallas_call(
        _ece_kernel,
        in_specs=[
            pl.BlockSpec(memory_space=pltpu.MemorySpace.VMEM),
            pl.BlockSpec(memory_space=pltpu.MemorySpace.VMEM),
            pl.BlockSpec(memory_space=pltpu.MemorySpace.HBM),
            pl.BlockSpec(memory_space=pltpu.MemorySpace.HBM),
        ],
        out_specs=pl.BlockSpec(memory_space=pltpu.MemorySpace.VMEM),
        out_shape=jax.ShapeDtypeStruct((8, 128), jnp.float32),
        scratch_shapes=[
            pltpu.VMEM((NBUF, r, c), jnp.float32),
            pltpu.VMEM((NBUF, r, 1), jnp.int32),
            pltpu.SemaphoreType.DMA((NBUF,)),
            pltpu.SemaphoreType.DMA((NBUF,)),
        ],
    )(lo, hi, probs, labels2)

    ece = out[0, 0:1]
    correct = out[1, 0:N_BINS]
    num = out[2, 0:N_BINS]
    return (ece, correct, num)
